# baseline (speedup 1.0000x reference)
import functools

import jax
import jax.numpy as jnp
from jax import lax
from jax.experimental import pallas as pl
from jax.experimental.pallas import tpu as pltpu
from jax.experimental.pallas import tpu_sc as plsc

VOCAB = 30522
EMB = 128
EMBW = EMB // 2
HID = 512
B = 128
S = 512
SH = S // 2
EPS = 1e-12

NC = 2
NS = 16
NW = NC * NS

N_ROWS = B * S
ROWS_PER_W = N_ROWS // NW
CHUNK = 128
PCHUNK = CHUNK // 2
N_CHUNKS = ROWS_PER_W // CHUNK


def _sc_gather_pack(table_hbm, idx_hbm, out_hbm, idx_v, rows_v, bf_v,
                    gsem0, gsem1, ssem0, ssem1):
    wid = lax.axis_index("s") * NC + lax.axis_index("c")
    base = wid * ROWS_PER_W
    pbase = wid * (ROWS_PER_W // 2)
    pltpu.sync_copy(idx_hbm.at[pl.ds(base, ROWS_PER_W)], idx_v)
    gsems = (gsem0, gsem1)
    ssems = (ssem0, ssem1)

    def gather_start(j, bb):
        return pltpu.async_copy(
            table_hbm.at[idx_v.at[pl.ds(j * CHUNK, CHUNK)]], rows_v.at[bb], gsems[bb]
        )

    def convert(bb):
        rows = rows_v.at[bb]
        bf = bf_v.at[bb]

        def body(kk, carry):
            rb = pl.multiple_of(16 * kk, 16)
            pb = pl.multiple_of(8 * kk, 8)
            for q in range(8):
                for half in range(2):
                    for i in range(4):
                        a = rows[rb + 2 * q + half, pl.ds(16 * i, 16)]
                        b2 = rows[rb + 2 * q + half, pl.ds(64 + 16 * i, 16)]
                        w = ((a >> 16) & 0xFFFF) | (b2 & -65536)
                        bf[pb + q, pl.ds(half * 64 + 16 * i, 16)] = w
            return carry

        lax.fori_loop(0, PCHUNK // 8, body, 0)

    g = [gather_start(0, 0), None]
    scat = [None, None]
    for j in range(N_CHUNKS):
        b = j & 1
        g[b].wait()
        if j + 1 < N_CHUNKS:
            g[1 - b] = gather_start(j + 1, 1 - b)
        if scat[b] is not None:
            scat[b].wait()
        convert(b)
        scat[b] = pltpu.async_copy(
            bf_v.at[b], out_hbm.at[pl.ds(pbase + j * PCHUNK, PCHUNK)], ssems[b]
        )
    for b in (0, 1):
        if scat[b] is not None:
            scat[b].wait()


def _gather_rows(table_i32, ids):
    gather = functools.partial(
        pl.kernel,
        out_type=jax.ShapeDtypeStruct((N_ROWS // 2, EMB), jnp.int32),
        mesh=plsc.VectorSubcoreMesh(
            core_axis_name="c", subcore_axis_name="s", num_cores=NC
        ),
        scratch_types=[
            pltpu.VMEM((ROWS_PER_W,), jnp.int32),
            pltpu.VMEM((2, CHUNK, EMB), jnp.int32),
            pltpu.VMEM((2, PCHUNK, EMB), jnp.int32),
            pltpu.SemaphoreType.DMA,
            pltpu.SemaphoreType.DMA,
            pltpu.SemaphoreType.DMA,
            pltpu.SemaphoreType.DMA,
        ],
    )(_sc_gather_pack)
    return gather(table_i32, ids)


BG = 8


def _tc_dense(e_ref, tt_ref, posb_ref, te_ref, gam_ref, bet_ref, w_ref, out_ref):
    u = e_ref[...]
    ua, ub = u[:, :, :EMBW], u[:, :, EMBW:]
    m = jnp.int32(-65536)

    def unpack(v):
        return jnp.concatenate(
            [pltpu.bitcast(v << 16, jnp.float32),
             pltpu.bitcast(v & m, jnp.float32)], axis=2)

    e = jnp.concatenate([unpack(ua), unpack(ub)], axis=1)
    z = jnp.zeros((BG, 1, EMB), jnp.float32)
    left = jnp.concatenate([e[:, 1:, :], z], axis=1)
    right = jnp.concatenate([z, e[:, :-1, :]], axis=1)
    tri = jnp.concatenate([left, e, right], axis=2).reshape(BG * S, 3 * EMB)
    x = jnp.dot(tri, w_ref[...], preferred_element_type=jnp.float32)
    x = x.reshape(BG, S, HID)
    te = te_ref[...]
    tt = tt_ref[...]
    typ = te[0][None, None, :] + tt[:, :, None] * (te[1] - te[0])[None, None, :]
    emb = x + posb_ref[...][None, :, :] + typ
    mean = jnp.mean(emb, axis=-1, keepdims=True)
    cen = emb - mean
    var = jnp.mean(cen * cen, axis=-1, keepdims=True)
    norm = cen * lax.rsqrt(var + EPS)
    out_ref[...] = norm * gam_ref[...][0][None, None, :] + bet_ref[...][0][None, None, :]


def kernel(input_ids, token_type_ids, word_emb, pos_emb, type_emb, W, b, gamma, beta):
    ids_r = jnp.swapaxes(
        input_ids.astype(jnp.int32).reshape(B, 2, SH), 1, 2
    ).reshape(N_ROWS)
    table_i32 = jax.lax.bitcast_convert_type(word_emb, jnp.int32)
    e_packed = _gather_rows(table_i32, ids_r).reshape(B, SH, EMB)

    tt_f = token_type_ids.astype(jnp.float32)
    posb = pos_emb + b[None, :]
    gam = gamma.reshape(1, HID)
    bet = beta.reshape(1, HID)

    grid = (B // BG,)
    out = pl.pallas_call(
        _tc_dense,
        grid=grid,
        in_specs=[
            pl.BlockSpec((BG, SH, EMB), lambda i: (i, 0, 0)),
            pl.BlockSpec((BG, S), lambda i: (i, 0)),
            pl.BlockSpec((S, HID), lambda i: (0, 0)),
            pl.BlockSpec((2, HID), lambda i: (0, 0)),
            pl.BlockSpec((1, HID), lambda i: (0, 0)),
            pl.BlockSpec((1, HID), lambda i: (0, 0)),
            pl.BlockSpec((3 * EMB, HID), lambda i: (0, 0)),
        ],
        out_specs=pl.BlockSpec((BG, S, HID), lambda i: (i, 0, 0)),
        out_shape=jax.ShapeDtypeStruct((B, S, HID), jnp.float32),
    )(e_packed, tt_f, posb, type_emb, gam, bet, W)
    return out


# speedup vs baseline: 1.2844x; 1.2844x over previous
import functools

import jax
import jax.numpy as jnp
from jax import lax
from jax.experimental import pallas as pl
from jax.experimental.pallas import tpu as pltpu
from jax.experimental.pallas import tpu_sc as plsc

VOCAB = 30522
EMB = 128
EMBW = EMB // 2
HID = 512
B = 128
S = 512
EPS = 1e-12

NC = 2
NS = 16
NW = NC * NS

G = 4
BG_ROWS = B // G
GROUP_ROWS = BG_ROWS * S
ROWS_PER_W = GROUP_ROWS // NW
CHUNK = 128
PCHUNK = CHUNK // 2
HC = CHUNK // 2
N_CHUNKS = ROWS_PER_W // CHUNK


def _sc_gather_pack(table_hbm, idx_hbm, out_hbm, idx_v, rows_v, bf_v,
                    gsem0, gsem1, ssem0, ssem1):
    wid = lax.axis_index("s") * NC + lax.axis_index("c")
    base = wid * ROWS_PER_W
    pbase = wid * (ROWS_PER_W // 2)
    pltpu.sync_copy(idx_hbm.at[pl.ds(base, ROWS_PER_W)], idx_v)
    gsems = (gsem0, gsem1)
    ssems = (ssem0, ssem1)

    def gather_start(j, bb):
        return pltpu.async_copy(
            table_hbm.at[idx_v.at[pl.ds(j * CHUNK, CHUNK)]], rows_v.at[bb], gsems[bb]
        )

    def convert(bb):
        rows = rows_v.at[bb]
        bf = bf_v.at[bb]

        def body(kk, carry):
            rb = pl.multiple_of(8 * kk, 8)
            for q in range(8):
                for half in range(2):
                    r = rb + q + half * HC
                    for i in range(4):
                        a = rows[r, pl.ds(16 * i, 16)]
                        b2 = rows[r, pl.ds(64 + 16 * i, 16)]
                        w = ((a >> 16) & 0xFFFF) | (b2 & -65536)
                        bf[rb + q, pl.ds(half * 64 + 16 * i, 16)] = w
            return carry

        lax.fori_loop(0, PCHUNK // 8, body, 0)

    g = [gather_start(0, 0), None]
    scat = [None, None]
    for j in range(N_CHUNKS):
        b = j & 1
        g[b].wait()
        if j + 1 < N_CHUNKS:
            g[1 - b] = gather_start(j + 1, 1 - b)
        if scat[b] is not None:
            scat[b].wait()
        convert(b)
        scat[b] = pltpu.async_copy(
            bf_v.at[b], out_hbm.at[pl.ds(pbase + j * PCHUNK, PCHUNK)], ssems[b]
        )
    for b in (0, 1):
        if scat[b] is not None:
            scat[b].wait()


def _gather_rows(table_i32, ids):
    gather = functools.partial(
        pl.kernel,
        out_type=jax.ShapeDtypeStruct((GROUP_ROWS // 2, EMB), jnp.int32),
        mesh=plsc.VectorSubcoreMesh(
            core_axis_name="c", subcore_axis_name="s", num_cores=NC
        ),
        scratch_types=[
            pltpu.VMEM((ROWS_PER_W,), jnp.int32),
            pltpu.VMEM((2, CHUNK, EMB), jnp.int32),
            pltpu.VMEM((2, PCHUNK, EMB), jnp.int32),
            pltpu.SemaphoreType.DMA,
            pltpu.SemaphoreType.DMA,
            pltpu.SemaphoreType.DMA,
            pltpu.SemaphoreType.DMA,
        ],
    )(_sc_gather_pack)
    return gather(table_i32, ids)


BG = 8
STEPS_PER_G = BG_ROWS // BG
LPC = S // CHUNK


def _tc_dense(e_ref, tt_ref, posb_ref, te_ref, gam_ref, bet_ref, w_ref, out_ref):
    u = e_ref[...]
    m = jnp.int32(-65536)

    def unpack(v):
        return jnp.concatenate(
            [pltpu.bitcast(v << 16, jnp.float32),
             pltpu.bitcast(v & m, jnp.float32)], axis=2)

    pieces = []
    for cc in range(LPC):
        lines = u[:, HC * cc:HC * (cc + 1), :]
        pieces.append(unpack(lines[:, :, :EMBW]))
        pieces.append(unpack(lines[:, :, EMBW:]))
    e = jnp.concatenate(pieces, axis=1)
    z = jnp.zeros((BG, 1, EMB), jnp.float32)
    left = jnp.concatenate([e[:, 1:, :], z], axis=1)
    right = jnp.concatenate([z, e[:, :-1, :]], axis=1)
    tri = jnp.concatenate([left, e, right], axis=2).reshape(BG * S, 3 * EMB)
    x = jnp.dot(tri, w_ref[...], preferred_element_type=jnp.float32)
    x = x.reshape(BG, S, HID)
    te = te_ref[...]
    tt = tt_ref[...]
    typ = te[0][None, None, :] + tt[:, :, None] * (te[1] - te[0])[None, None, :]
    emb = x + posb_ref[...][None, :, :] + typ
    mean = jnp.mean(emb, axis=-1, keepdims=True)
    cen = emb - mean
    var = jnp.mean(cen * cen, axis=-1, keepdims=True)
    norm = cen * lax.rsqrt(var + EPS)
    out_ref[...] = norm * gam_ref[...][0][None, None, :] + bet_ref[...][0][None, None, :]


def _tc_dense_alias(e_ref, tt_ref, posb_ref, te_ref, gam_ref, bet_ref, w_ref,
                    buf_ref, out_ref):
    del buf_ref
    _tc_dense(e_ref, tt_ref, posb_ref, te_ref, gam_ref, bet_ref, w_ref, out_ref)


def _dense_group(g, e_g, tt_f, posb, type_emb, gam, bet, W, buf):
    base_specs = [
        pl.BlockSpec((BG, S // 2, EMB), lambda i: (i, 0, 0)),
        pl.BlockSpec((BG, S), lambda i, g=g: (g * STEPS_PER_G + i, 0)),
        pl.BlockSpec((S, HID), lambda i: (0, 0)),
        pl.BlockSpec((2, HID), lambda i: (0, 0)),
        pl.BlockSpec((1, HID), lambda i: (0, 0)),
        pl.BlockSpec((1, HID), lambda i: (0, 0)),
        pl.BlockSpec((3 * EMB, HID), lambda i: (0, 0)),
    ]
    out_spec = pl.BlockSpec((BG, S, HID), lambda i, g=g: (g * STEPS_PER_G + i, 0, 0))
    out_shape = jax.ShapeDtypeStruct((B, S, HID), jnp.float32)
    args = (e_g, tt_f, posb, type_emb, gam, bet, W)
    if buf is None:
        return pl.pallas_call(
            _tc_dense,
            grid=(STEPS_PER_G,),
            in_specs=base_specs,
            out_specs=out_spec,
            out_shape=out_shape,
        )(*args)
    return pl.pallas_call(
        _tc_dense_alias,
        grid=(STEPS_PER_G,),
        in_specs=base_specs + [pl.BlockSpec(memory_space=pl.ANY)],
        out_specs=out_spec,
        out_shape=out_shape,
        input_output_aliases={7: 0},
    )(*args, buf)


def kernel(input_ids, token_type_ids, word_emb, pos_emb, type_emb, W, b, gamma, beta):
    ids = input_ids.reshape(G, GROUP_ROWS).astype(jnp.int32)
    table_i32 = jax.lax.bitcast_convert_type(word_emb, jnp.int32)
    e_groups = [
        _gather_rows(table_i32, ids[g]).reshape(BG_ROWS, S // 2, EMB)
        for g in range(G)
    ]

    tt_f = token_type_ids.astype(jnp.float32)
    posb = pos_emb + b[None, :]
    gam = gamma.reshape(1, HID)
    bet = beta.reshape(1, HID)

    buf = None
    for g in range(G):
        buf = _dense_group(g, e_groups[g], tt_f, posb, type_emb, gam, bet, W, buf)
    return buf


# speedup vs baseline: 1.4478x; 1.1272x over previous
import functools

import jax
import jax.numpy as jnp
from jax import lax
from jax.experimental import pallas as pl
from jax.experimental.pallas import tpu as pltpu
from jax.experimental.pallas import tpu_sc as plsc

VOCAB = 30522
EMB = 128
HID = 512
B = 128
S = 512
EPS = 1e-12

NC = 2
NS = 16
NW = NC * NS

N_ROWS = B * S
ROWS_PER_W = N_ROWS // NW
CHUNK = 256
N_CHUNKS = ROWS_PER_W // CHUNK


def _sc_gather(table_hbm, idx_hbm, out_hbm, idx_v, rows_v, gsem0, gsem1, ssem0, ssem1):
    wid = lax.axis_index("s") * NC + lax.axis_index("c")
    base = wid * ROWS_PER_W
    pltpu.sync_copy(idx_hbm.at[pl.ds(base, ROWS_PER_W)], idx_v)
    gsems = (gsem0, gsem1)
    ssems = (ssem0, ssem1)

    def gather_start(j, bb):
        return pltpu.async_copy(
            table_hbm.at[idx_v.at[pl.ds(j * CHUNK, CHUNK)]], rows_v.at[bb], gsems[bb]
        )

    g = [gather_start(0, 0), None]
    scat = [None, None]
    for j in range(N_CHUNKS):
        b = j & 1
        if j + 1 < N_CHUNKS:
            if scat[1 - b] is not None:
                scat[1 - b].wait()
            g[1 - b] = gather_start(j + 1, 1 - b)
        g[b].wait()
        scat[b] = pltpu.async_copy(
            rows_v.at[b], out_hbm.at[pl.ds(base + j * CHUNK, CHUNK)], ssems[b]
        )
    for b in (0, 1):
        if scat[b] is not None:
            scat[b].wait()


def _gather_rows(table, ids):
    gather = functools.partial(
        pl.kernel,
        out_type=jax.ShapeDtypeStruct((N_ROWS, EMB), jnp.float32),
        mesh=plsc.VectorSubcoreMesh(
            core_axis_name="c", subcore_axis_name="s", num_cores=NC
        ),
        scratch_types=[
            pltpu.VMEM((ROWS_PER_W,), jnp.int32),
            pltpu.VMEM((2, CHUNK, EMB), jnp.float32),
            pltpu.SemaphoreType.DMA,
            pltpu.SemaphoreType.DMA,
            pltpu.SemaphoreType.DMA,
            pltpu.SemaphoreType.DMA,
        ],
    )(_sc_gather)
    return gather(table, ids)


BG = 8


def _tc_dense(e_ref, tt_ref, posb_ref, te_ref, gam_ref, bet_ref, w_ref, out_ref):
    e = e_ref[...]
    z = jnp.zeros((BG, 1, EMB), jnp.float32)
    left = jnp.concatenate([e[:, 1:, :], z], axis=1)
    right = jnp.concatenate([z, e[:, :-1, :]], axis=1)
    tri = jnp.concatenate([left, e, right], axis=2).reshape(BG * S, 3 * EMB)
    x = jnp.dot(tri, w_ref[...], preferred_element_type=jnp.float32)
    x = x.reshape(BG, S, HID)
    te = te_ref[...]
    tt = tt_ref[...]
    typ = te[0][None, None, :] + tt[:, :, None] * (te[1] - te[0])[None, None, :]
    emb = x + posb_ref[...][None, :, :] + typ
    mean = jnp.mean(emb, axis=-1, keepdims=True)
    cen = emb - mean
    var = jnp.mean(cen * cen, axis=-1, keepdims=True)
    norm = cen * lax.rsqrt(var + EPS)
    out_ref[...] = norm * gam_ref[...][0][None, None, :] + bet_ref[...][0][None, None, :]


def kernel(input_ids, token_type_ids, word_emb, pos_emb, type_emb, W, b, gamma, beta):
    ids = input_ids.reshape(-1).astype(jnp.int32)
    e = _gather_rows(word_emb, ids).reshape(B, S, EMB)

    tt_f = token_type_ids.astype(jnp.float32)
    posb = pos_emb + b[None, :]
    gam = gamma.reshape(1, HID)
    bet = beta.reshape(1, HID)

    grid = (B // BG,)
    out = pl.pallas_call(
        _tc_dense,
        grid=grid,
        in_specs=[
            pl.BlockSpec((BG, S, EMB), lambda i: (i, 0, 0)),
            pl.BlockSpec((BG, S), lambda i: (i, 0)),
            pl.BlockSpec((S, HID), lambda i: (0, 0)),
            pl.BlockSpec((2, HID), lambda i: (0, 0)),
            pl.BlockSpec((1, HID), lambda i: (0, 0)),
            pl.BlockSpec((1, HID), lambda i: (0, 0)),
            pl.BlockSpec((3 * EMB, HID), lambda i: (0, 0)),
        ],
        out_specs=pl.BlockSpec((BG, S, HID), lambda i: (i, 0, 0)),
        out_shape=jax.ShapeDtypeStruct((B, S, HID), jnp.float32),
    )(e, tt_f, posb, type_emb, gam, bet, W)
    return out
